# baseline (speedup 1.0000x reference)
import functools

import jax
import jax.numpy as jnp
from jax.experimental import pallas as pl
from jax.experimental.pallas import tpu as pltpu

N = 4096
BLOCK = 256


def _fused_kernel(sr_ref, sp_ref, h0r_ref, h0p_ref,
                  w0_ref, sw0_ref, w1_ref, sw1_ref,
                  out1r_ref, out1p_ref,
                  vu_ref, vv_ref, self_r_ref, self_p_ref,
                  h1r_ref, h1p_ref, *, block):
    l = pl.program_id(0)
    i = pl.program_id(1)
    rows = pl.ds(i * block, block)

    @pl.when(jnp.logical_and(l == 0, i == 0))
    def _init0():
        hr = h0r_ref[...]
        hp = h0p_ref[...]
        w0 = w0_ref[0]
        w1 = w0_ref[1]
        sw = sw0_ref[...]
        vu_ref[...] = jnp.concatenate(
            [jnp.dot(hr, w0, preferred_element_type=jnp.float32),
             jnp.dot(hr, w1, preferred_element_type=jnp.float32)],
            axis=1).astype(jnp.bfloat16)
        vv_ref[...] = jnp.concatenate(
            [jnp.dot(hp, w0, preferred_element_type=jnp.float32),
             jnp.dot(hp, w1, preferred_element_type=jnp.float32)],
            axis=1).astype(jnp.bfloat16)
        self_r_ref[...] = jnp.dot(hr, sw, preferred_element_type=jnp.float32)
        self_p_ref[...] = jnp.dot(hp, sw, preferred_element_type=jnp.float32)

    @pl.when(jnp.logical_and(l == 1, i == 0))
    def _init1():
        hr = h1r_ref[...]
        hp = h1p_ref[...]
        w0 = w1_ref[0]
        w1 = w1_ref[1]
        sw = sw1_ref[...]
        vu_ref[:, :32] = jnp.concatenate(
            [jnp.dot(hr, w0, preferred_element_type=jnp.float32),
             jnp.dot(hr, w1, preferred_element_type=jnp.float32)],
            axis=1).astype(jnp.bfloat16)
        vv_ref[:, :32] = jnp.concatenate(
            [jnp.dot(hp, w0, preferred_element_type=jnp.float32),
             jnp.dot(hp, w1, preferred_element_type=jnp.float32)],
            axis=1).astype(jnp.bfloat16)
        self_r_ref[:, :32] = jnp.dot(hr, sw,
                                     preferred_element_type=jnp.float32)
        self_p_ref[:, :32] = jnp.dot(hp, sw,
                                     preferred_element_type=jnp.float32)

    sr0 = sr_ref[0].astype(jnp.bfloat16)
    sr1 = sr_ref[1].astype(jnp.bfloat16)
    sp0 = sp_ref[0].astype(jnp.bfloat16)
    sp1 = sp_ref[1].astype(jnp.bfloat16)

    @pl.when(l == 0)
    def _body0():
        k = 32
        vu = vu_ref[...]
        vv = vv_ref[...]
        agg_r = jnp.concatenate(
            [jnp.dot(sr0, vv[:, :k], preferred_element_type=jnp.float32),
             jnp.dot(sr1, vv[:, k:], preferred_element_type=jnp.float32)],
            axis=1)
        agg_p = jnp.concatenate(
            [jnp.dot(sp0, vu[:, :k], preferred_element_type=jnp.float32),
             jnp.dot(sp1, vu[:, k:], preferred_element_type=jnp.float32)],
            axis=1)
        h1r_ref[rows, :] = jnp.maximum(agg_r + self_r_ref[rows, :], 0.0)
        h1p_ref[rows, :] = jnp.maximum(agg_p + self_p_ref[rows, :], 0.0)

    @pl.when(l == 1)
    def _body1():
        k = 16
        vu = vu_ref[:, :32]
        vv = vv_ref[:, :32]
        agg_r = jnp.concatenate(
            [jnp.dot(sr0, vv[:, :k], preferred_element_type=jnp.float32),
             jnp.dot(sr1, vv[:, k:], preferred_element_type=jnp.float32)],
            axis=1)
        agg_p = jnp.concatenate(
            [jnp.dot(sp0, vu[:, :k], preferred_element_type=jnp.float32),
             jnp.dot(sp1, vu[:, k:], preferred_element_type=jnp.float32)],
            axis=1)
        out1r_ref[...] = jnp.maximum(agg_r + self_r_ref[rows, :32], 0.0)
        out1p_ref[...] = jnp.maximum(agg_p + self_p_ref[rows, :32], 0.0)


def kernel(RNA_supports, protein_supports, RNA_inputs, protein_inputs,
           W0, W1, SW0, SW1):
    block = BLOCK
    nblk = N // block
    kern = functools.partial(_fused_kernel, block=block)
    sup_spec = pl.BlockSpec((2, block, N), lambda l, i: (0, i, 0))
    full2 = lambda l, i: (0, 0)
    full3 = lambda l, i: (0, 0, 0)
    out = pl.pallas_call(
        kern,
        grid_spec=pltpu.PrefetchScalarGridSpec(
            num_scalar_prefetch=0,
            grid=(2, nblk),
            in_specs=[
                sup_spec,
                sup_spec,
                pl.BlockSpec((N, 128), full2),
                pl.BlockSpec((N, 128), full2),
                pl.BlockSpec((2, 128, 32), full3),
                pl.BlockSpec((128, 64), full2),
                pl.BlockSpec((2, 64, 16), full3),
                pl.BlockSpec((64, 32), full2),
            ],
            out_specs=[
                pl.BlockSpec((block, 32), lambda l, i: (i, 0)),
                pl.BlockSpec((block, 32), lambda l, i: (i, 0)),
            ],
            scratch_shapes=[
                pltpu.VMEM((N, 64), jnp.bfloat16),
                pltpu.VMEM((N, 64), jnp.bfloat16),
                pltpu.VMEM((N, 64), jnp.float32),
                pltpu.VMEM((N, 64), jnp.float32),
                pltpu.VMEM((N, 64), jnp.float32),
                pltpu.VMEM((N, 64), jnp.float32),
            ],
        ),
        out_shape=[
            jax.ShapeDtypeStruct((N, 32), jnp.float32),
            jax.ShapeDtypeStruct((N, 32), jnp.float32),
        ],
        compiler_params=pltpu.CompilerParams(
            dimension_semantics=("arbitrary", "arbitrary"),
        ),
    )(RNA_supports, protein_supports, RNA_inputs, protein_inputs,
      W0, SW0, W1, SW1)
    return (out[0], out[1])
